# baseline (speedup 1.0000x reference)
import functools

import jax
import jax.numpy as jnp
from jax import lax
from jax.experimental import pallas as pl
from jax.experimental.pallas import tpu as pltpu, tpu_sc as plsc

N = 10000
E = 320000
D = 128
H = 16
C = 5
CP = 8

NPAD = 10240
NC = 2
NS = 16
NW = NC * NS
NITER = 4
CHUNK = 2512
EWP = NITER * CHUNK
EP = NW * EWP
RPT = NPAD // NS


def _fill(ref, n, val):
    v = jnp.full((16,), val, jnp.float32)

    def body(i, c):
        ref[pl.ds(i * 16, 16)] = v
        return c

    lax.fori_loop(0, n // 16, body, 0)


def _make_deg_kernel():
    mesh = plsc.VectorSubcoreMesh(core_axis_name="c", subcore_axis_name="s")

    @functools.partial(
        pl.kernel,
        mesh=mesh,
        out_type=jax.ShapeDtypeStruct((NC, NPAD), jnp.float32),
        scratch_types=[
            pltpu.VMEM((EWP,), jnp.int32),
            pltpu.VMEM((EWP,), jnp.float32),
            pltpu.VMEM((RPT,), jnp.float32),
            pltpu.VMEM_SHARED((NPAD,), jnp.float32),
            pltpu.SemaphoreType.DMA,
        ],
    )
    def deg_kernel(ei, out, dst_v, ones_v, z_v, acc, sem):
        cid = lax.axis_index("c")
        sid = lax.axis_index("s")
        wid = cid * NS + sid
        base = wid * EWP

        idx_cp = pltpu.async_copy(ei.at[pl.ds(EP + base, EWP)], dst_v, sem)
        _fill(z_v, RPT, 0.0)
        _fill(ones_v, EWP, 1.0)
        pltpu.sync_copy(z_v, acc.at[pl.ds(sid * RPT, RPT)])
        plsc.subcore_barrier()
        idx_cp.wait()
        pltpu.sync_copy(ones_v, acc.at[dst_v], add=True)
        plsc.subcore_barrier()
        pltpu.sync_copy(
            acc.at[pl.ds(sid * RPT, RPT)],
            out.at[cid, pl.ds(sid * RPT, RPT)],
        )

    return deg_kernel


def _make_edge_kernel(width):
    mesh = plsc.VectorSubcoreMesh(core_axis_name="c", subcore_axis_name="s")

    @functools.partial(
        pl.kernel,
        mesh=mesh,
        out_type=jax.ShapeDtypeStruct((NC, NPAD, width), jnp.float32),
        scratch_types=[
            [pltpu.VMEM((CHUNK,), jnp.int32) for _ in range(NITER)],
            [pltpu.VMEM((CHUNK,), jnp.int32) for _ in range(NITER)],
            [pltpu.VMEM((CHUNK, width), jnp.float32) for _ in range(2)],
            pltpu.VMEM_SHARED((NPAD, width), jnp.float32),
            pltpu.SemaphoreType.DMA,
            [pltpu.SemaphoreType.DMA for _ in range(2)],
        ],
        compiler_params=pltpu.CompilerParams(use_tc_tiling_on_sc=False),
    )
    def edge_kernel(ei, g, zeros, out, srcs, dsts, rows, acc, semi, semg):
        cid = lax.axis_index("c")
        sid = lax.axis_index("s")
        wid = cid * NS + sid
        base = wid * EWP

        cps = []
        cps.append(pltpu.async_copy(
            zeros.at[pl.ds(sid * RPT, RPT)],
            acc.at[pl.ds(sid * RPT, RPT)], semi))
        for j in range(NITER):
            off = base + j * CHUNK
            cps.append(pltpu.async_copy(ei.at[pl.ds(off, CHUNK)],
                                        srcs[j], semi))
            cps.append(pltpu.async_copy(ei.at[pl.ds(EP + off, CHUNK)],
                                        dsts[j], semi))
        for cp in cps:
            cp.wait()
        plsc.subcore_barrier()

        gathers = [None, None]
        gathers[0] = pltpu.async_copy(g.at[srcs[0]], rows[0], semg[0])
        for j in range(NITER):
            gathers[j % 2].wait()
            if j + 1 < NITER:
                nb = (j + 1) % 2
                gathers[nb] = pltpu.async_copy(
                    g.at[srcs[j + 1]], rows[nb], semg[nb])
            pltpu.sync_copy(rows[j % 2], acc.at[dsts[j]], add=True)

        plsc.subcore_barrier()
        pltpu.sync_copy(
            acc.at[pl.ds(sid * RPT, RPT)],
            out.at[cid, pl.ds(sid * RPT, RPT)],
        )

    return edge_kernel


def _k2_body(x_ref, w1_ref, degp_ref, g1_ref, dinv_ref):
    deg = degp_ref[0:N, :] + degp_ref[NPAD:NPAD + N, :] + 1.0
    dinv = lax.rsqrt(deg)
    h1 = jnp.dot(x_ref[...], w1_ref[...], preferred_element_type=jnp.float32)
    g1_ref[pl.ds(0, N), :] = h1 * dinv
    g1_ref[pl.ds(N, NPAD - N), :] = jnp.zeros((NPAD - N, H), jnp.float32)
    dinv_ref[...] = dinv


def _k4_body(p_ref, g1_ref, dinv_ref, b1_ref, w2_ref, g2_ref):
    dinv = dinv_ref[...]
    s = p_ref[0:N, :] + p_ref[NPAD:NPAD + N, :] + g1_ref[0:N, :]
    r1 = jnp.maximum(dinv * s + b1_ref[...], 0.0)
    h2 = jnp.dot(r1, w2_ref[...], preferred_element_type=jnp.float32)
    g2_ref[pl.ds(0, N), :] = h2 * dinv
    g2_ref[pl.ds(N, NPAD - N), :] = jnp.zeros((NPAD - N, CP), jnp.float32)


def _k6_body(q_ref, g2_ref, dinv_ref, b2_ref, w3_ref, b3_ref, w4_ref, b4_ref,
             out_ref):
    dinv = dinv_ref[...]
    s = q_ref[0:N, :] + q_ref[NPAD:NPAD + N, :] + g2_ref[0:N, :]
    agg2 = dinv * s + b2_ref[...]
    z1 = jnp.maximum(
        jnp.dot(agg2, w3_ref[...], preferred_element_type=jnp.float32)
        + b3_ref[...], 0.0)
    z = jnp.dot(z1, w4_ref[...], preferred_element_type=jnp.float32) \
        + b4_ref[...]
    m = jnp.max(z, axis=1, keepdims=True)
    lse = jnp.log(jnp.sum(jnp.exp(z - m), axis=1, keepdims=True)) + m
    out_ref[...] = z - lse


def kernel(x, edge_index, W1, b1, W2, b2, W3, b3, W4, b4):
    deg_k = _make_deg_kernel()
    edge16 = _make_edge_kernel(H)
    edge8 = _make_edge_kernel(CP)

    pad = jnp.full((EP - E,), N, jnp.int32)
    ei_flat = jnp.concatenate(
        [edge_index[0], pad, edge_index[1], pad])

    degp = deg_k(ei_flat).reshape(NC * NPAD, 1)

    g1, dinv = pl.pallas_call(
        _k2_body,
        out_shape=[
            jax.ShapeDtypeStruct((NPAD, H), jnp.float32),
            jax.ShapeDtypeStruct((N, 1), jnp.float32),
        ],
    )(x, W1, degp)

    z16 = jnp.zeros((NPAD, H), jnp.float32)
    p1 = edge16(ei_flat, g1, z16).reshape(NC * NPAD, H)

    W2p = jnp.concatenate([W2, jnp.zeros((H, CP - C), jnp.float32)], axis=1)
    g2 = pl.pallas_call(
        _k4_body,
        out_shape=jax.ShapeDtypeStruct((NPAD, CP), jnp.float32),
    )(p1, g1, dinv, b1.reshape(1, H), W2p)

    z8 = jnp.zeros((NPAD, CP), jnp.float32)
    p2 = edge8(ei_flat, g2, z8).reshape(NC * NPAD, CP)

    b2p = jnp.concatenate([b2, jnp.zeros((CP - C,), jnp.float32)])
    W3p = jnp.concatenate([W3, jnp.zeros((CP - C, 32), jnp.float32)], axis=0)
    out = pl.pallas_call(
        _k6_body,
        out_shape=jax.ShapeDtypeStruct((N, C), jnp.float32),
    )(p2, g2, dinv, b2p.reshape(1, CP), W3p, b3.reshape(1, 32), W4,
      b4.reshape(1, C))
    return out


# speedup vs baseline: 1.1873x; 1.1873x over previous
import functools

import jax
import jax.numpy as jnp
from jax import lax
from jax.experimental import pallas as pl
from jax.experimental.pallas import tpu as pltpu, tpu_sc as plsc

N = 10000
E = 320000
D = 128
H = 16
C = 5
CP = 8

NPAD = 10240
NC = 2
NS = 16
NW = NC * NS
EW = E // NW
CHUNKS = (2504, 2504, 2504, 2488)
OFFS = (0, 2504, 5008, 7512)
NITER = len(CHUNKS)
CMAX = CHUNKS[0]
RPT = NPAD // NS


def _fill(ref, n, val):
    v = jnp.full((16,), val, jnp.float32)

    def body(i, c):
        ref[pl.ds(i * 16, 16)] = v
        return c

    lax.fori_loop(0, n // 16, body, 0)


def _make_deg_kernel():
    mesh = plsc.VectorSubcoreMesh(core_axis_name="c", subcore_axis_name="s")

    @functools.partial(
        pl.kernel,
        mesh=mesh,
        out_type=jax.ShapeDtypeStruct((NC, NPAD), jnp.float32),
        scratch_types=[
            pltpu.VMEM((EW,), jnp.int32),
            pltpu.VMEM((EW,), jnp.float32),
            pltpu.VMEM((RPT,), jnp.float32),
            pltpu.VMEM_SHARED((NPAD,), jnp.float32),
            pltpu.SemaphoreType.DMA,
        ],
    )
    def deg_kernel(ei, out, dst_v, ones_v, z_v, acc, sem):
        cid = lax.axis_index("c")
        sid = lax.axis_index("s")
        wid = cid * NS + sid
        base = wid * EW

        idx_cp = pltpu.async_copy(ei.at[pl.ds(E + base, EW)], dst_v, sem)
        _fill(z_v, RPT, 0.0)
        _fill(ones_v, EW, 1.0)
        pltpu.sync_copy(z_v, acc.at[pl.ds(sid * RPT, RPT)])
        plsc.subcore_barrier()
        idx_cp.wait()
        pltpu.sync_copy(ones_v, acc.at[dst_v], add=True)
        plsc.subcore_barrier()
        pltpu.sync_copy(
            acc.at[pl.ds(sid * RPT, RPT)],
            out.at[cid, pl.ds(sid * RPT, RPT)],
        )

    return deg_kernel


def _make_edge_kernel(width):
    mesh = plsc.VectorSubcoreMesh(core_axis_name="c", subcore_axis_name="s")

    @functools.partial(
        pl.kernel,
        mesh=mesh,
        out_type=jax.ShapeDtypeStruct((NC, NPAD, width), jnp.float32),
        scratch_types=[
            [pltpu.VMEM((c,), jnp.int32) for c in CHUNKS],
            [pltpu.VMEM((c,), jnp.int32) for c in CHUNKS],
            [pltpu.VMEM((CMAX, width), jnp.float32) for _ in range(2)],
            pltpu.VMEM_SHARED((NPAD, width), jnp.float32),
            pltpu.SemaphoreType.DMA,
            [pltpu.SemaphoreType.DMA for _ in range(2)],
        ],
        compiler_params=pltpu.CompilerParams(use_tc_tiling_on_sc=False),
    )
    def edge_kernel(ei, g, zeros, out, srcs, dsts, rows, acc, semi, semg):
        cid = lax.axis_index("c")
        sid = lax.axis_index("s")
        wid = cid * NS + sid
        base = wid * EW

        cps = []
        cps.append(pltpu.async_copy(
            zeros.at[pl.ds(sid * RPT, RPT)],
            acc.at[pl.ds(sid * RPT, RPT)], semi))
        for j in range(NITER):
            off = base + OFFS[j]
            cps.append(pltpu.async_copy(ei.at[pl.ds(off, CHUNKS[j])],
                                        srcs[j], semi))
            cps.append(pltpu.async_copy(ei.at[pl.ds(E + off, CHUNKS[j])],
                                        dsts[j], semi))
        for cp in cps:
            cp.wait()
        plsc.subcore_barrier()

        def buf(j):
            b = rows[j % 2]
            return b if CHUNKS[j] == CMAX else b.at[pl.ds(0, CHUNKS[j])]

        gathers = [None, None]
        gathers[0] = pltpu.async_copy(g.at[srcs[0]], buf(0), semg[0])
        for j in range(NITER):
            gathers[j % 2].wait()
            if j + 1 < NITER:
                nb = (j + 1) % 2
                gathers[nb] = pltpu.async_copy(
                    g.at[srcs[j + 1]], buf(j + 1), semg[nb])
            pltpu.sync_copy(buf(j), acc.at[dsts[j]], add=True)

        plsc.subcore_barrier()
        pltpu.sync_copy(
            acc.at[pl.ds(sid * RPT, RPT)],
            out.at[cid, pl.ds(sid * RPT, RPT)],
        )

    return edge_kernel


def _k2_body(x_ref, w1_ref, degp_ref, g1_ref, dinv_ref):
    deg = degp_ref[0:N, :] + degp_ref[NPAD:NPAD + N, :] + 1.0
    dinv = lax.rsqrt(deg)
    h1 = jnp.dot(x_ref[...], w1_ref[...], preferred_element_type=jnp.float32)
    g1_ref[pl.ds(0, N), :] = h1 * dinv
    g1_ref[pl.ds(N, NPAD - N), :] = jnp.zeros((NPAD - N, H), jnp.float32)
    dinv_ref[...] = dinv


def _k4_body(p_ref, g1_ref, dinv_ref, b1_ref, w2_ref, g2_ref):
    dinv = dinv_ref[...]
    s = p_ref[0:N, :] + p_ref[NPAD:NPAD + N, :] + g1_ref[0:N, :]
    r1 = jnp.maximum(dinv * s + b1_ref[...], 0.0)
    h2 = jnp.dot(r1, w2_ref[...], preferred_element_type=jnp.float32)
    g2_ref[pl.ds(0, N), :] = h2 * dinv
    g2_ref[pl.ds(N, NPAD - N), :] = jnp.zeros((NPAD - N, CP), jnp.float32)


def _k6_body(q_ref, g2_ref, dinv_ref, b2_ref, w3_ref, b3_ref, w4_ref, b4_ref,
             out_ref):
    dinv = dinv_ref[...]
    s = q_ref[0:N, :] + q_ref[NPAD:NPAD + N, :] + g2_ref[0:N, :]
    agg2 = dinv * s + b2_ref[...]
    z1 = jnp.maximum(
        jnp.dot(agg2, w3_ref[...], preferred_element_type=jnp.float32)
        + b3_ref[...], 0.0)
    z = jnp.dot(z1, w4_ref[...], preferred_element_type=jnp.float32) \
        + b4_ref[...]
    m = jnp.max(z, axis=1, keepdims=True)
    lse = jnp.log(jnp.sum(jnp.exp(z - m), axis=1, keepdims=True)) + m
    out_ref[...] = z - lse


def kernel(x, edge_index, W1, b1, W2, b2, W3, b3, W4, b4):
    deg_k = _make_deg_kernel()
    edge16 = _make_edge_kernel(H)
    edge8 = _make_edge_kernel(CP)

    ei_flat = edge_index.reshape(2 * E)

    degp = deg_k(ei_flat).reshape(NC * NPAD, 1)

    g1, dinv = pl.pallas_call(
        _k2_body,
        out_shape=[
            jax.ShapeDtypeStruct((NPAD, H), jnp.float32),
            jax.ShapeDtypeStruct((N, 1), jnp.float32),
        ],
    )(x, W1, degp)

    z16 = jnp.zeros((NPAD, H), jnp.float32)
    p1 = edge16(ei_flat, g1, z16).reshape(NC * NPAD, H)

    W2p = jnp.concatenate([W2, jnp.zeros((H, CP - C), jnp.float32)], axis=1)
    g2 = pl.pallas_call(
        _k4_body,
        out_shape=jax.ShapeDtypeStruct((NPAD, CP), jnp.float32),
    )(p1, g1, dinv, b1.reshape(1, H), W2p)

    z8 = jnp.zeros((NPAD, CP), jnp.float32)
    p2 = edge8(ei_flat, g2, z8).reshape(NC * NPAD, CP)

    b2p = jnp.concatenate([b2, jnp.zeros((CP - C,), jnp.float32)])
    W3p = jnp.concatenate([W3, jnp.zeros((CP - C, 32), jnp.float32)], axis=0)
    out = pl.pallas_call(
        _k6_body,
        out_shape=jax.ShapeDtypeStruct((N, C), jnp.float32),
    )(p2, g2, dinv, b2p.reshape(1, CP), W3p, b3.reshape(1, 32), W4,
      b4.reshape(1, C))
    return out


# speedup vs baseline: 1.1996x; 1.0103x over previous
import functools

import jax
import jax.numpy as jnp
from jax import lax
from jax.experimental import pallas as pl
from jax.experimental.pallas import tpu as pltpu, tpu_sc as plsc

N = 10000
E = 320000
D = 128
H = 16
C = 5
CP = 8

NPAD = 10240
NC = 2
NS = 16
NW = NC * NS
EW = E // NW
CHUNKS = (2504, 2504, 2504, 2488)
OFFS = (0, 2504, 5008, 7512)
NITER = len(CHUNKS)
CMAX = CHUNKS[0]
RPT = NPAD // NS


def _fill(ref, n, val):
    v = jnp.full((16,), val, jnp.float32)

    def body(i, c):
        ref[pl.ds(i * 16, 16)] = v
        return c

    lax.fori_loop(0, n // 16, body, 0)


def _make_deg_kernel():
    mesh = plsc.VectorSubcoreMesh(core_axis_name="c", subcore_axis_name="s")

    @functools.partial(
        pl.kernel,
        mesh=mesh,
        out_type=jax.ShapeDtypeStruct((NC, NPAD), jnp.float32),
        scratch_types=[
            pltpu.VMEM((EW,), jnp.int32),
            pltpu.VMEM((EW,), jnp.float32),
            pltpu.VMEM((RPT,), jnp.float32),
            pltpu.VMEM_SHARED((NPAD,), jnp.float32),
            pltpu.SemaphoreType.DMA,
        ],
    )
    def deg_kernel(ei, out, dst_v, ones_v, z_v, acc, sem):
        cid = lax.axis_index("c")
        sid = lax.axis_index("s")
        wid = cid * NS + sid
        base = wid * EW

        idx_cp = pltpu.async_copy(ei.at[pl.ds(E + base, EW)], dst_v, sem)
        _fill(z_v, RPT, 0.0)
        _fill(ones_v, EW, 1.0)
        pltpu.sync_copy(z_v, acc.at[pl.ds(sid * RPT, RPT)])
        plsc.subcore_barrier()
        idx_cp.wait()
        pltpu.sync_copy(ones_v, acc.at[dst_v], add=True)
        plsc.subcore_barrier()
        pltpu.sync_copy(
            acc.at[pl.ds(sid * RPT, RPT)],
            out.at[cid, pl.ds(sid * RPT, RPT)],
        )

    return deg_kernel


def _make_edge_kernel(width):
    mesh = plsc.VectorSubcoreMesh(core_axis_name="c", subcore_axis_name="s")

    @functools.partial(
        pl.kernel,
        mesh=mesh,
        out_type=jax.ShapeDtypeStruct((NC, NPAD, width), jnp.float32),
        scratch_types=[
            [pltpu.VMEM((c,), jnp.int32) for c in CHUNKS],
            [pltpu.VMEM((c,), jnp.int32) for c in CHUNKS],
            [pltpu.VMEM((CMAX, width), jnp.float32) for _ in range(2)],
            pltpu.VMEM_SHARED((NPAD, width), jnp.float32),
            pltpu.SemaphoreType.DMA,
            [pltpu.SemaphoreType.DMA for _ in range(2)],
            [pltpu.SemaphoreType.DMA for _ in range(2)],
        ],
        compiler_params=pltpu.CompilerParams(use_tc_tiling_on_sc=False),
    )
    def edge_kernel(ei, g, zeros, out, srcs, dsts, rows, acc, semi, semg,
                    sems):
        cid = lax.axis_index("c")
        sid = lax.axis_index("s")
        wid = cid * NS + sid
        base = wid * EW

        src0_cp = pltpu.async_copy(
            ei.at[pl.ds(base, CHUNKS[0])], srcs[0], semi)
        cps = [pltpu.async_copy(
            zeros.at[pl.ds(sid * RPT, RPT)],
            acc.at[pl.ds(sid * RPT, RPT)], semi)]
        for j in range(NITER):
            off = base + OFFS[j]
            if j > 0:
                cps.append(pltpu.async_copy(ei.at[pl.ds(off, CHUNKS[j])],
                                            srcs[j], semi))
            cps.append(pltpu.async_copy(ei.at[pl.ds(E + off, CHUNKS[j])],
                                        dsts[j], semi))

        def buf(j):
            b = rows[j % 2]
            return b if CHUNKS[j] == CMAX else b.at[pl.ds(0, CHUNKS[j])]

        src0_cp.wait()
        gathers = [None, None]
        gathers[0] = pltpu.async_copy(g.at[srcs[0]], buf(0), semg[0])
        for cp in cps:
            cp.wait()
        plsc.subcore_barrier()

        scat = [None, None]
        for j in range(NITER):
            gathers[j % 2].wait()
            if j + 1 < NITER:
                nb = (j + 1) % 2
                if scat[nb] is not None:
                    scat[nb].wait()
                    scat[nb] = None
                gathers[nb] = pltpu.async_copy(
                    g.at[srcs[j + 1]], buf(j + 1), semg[nb])
            scat[j % 2] = pltpu.async_copy(
                buf(j), acc.at[dsts[j]], sems[j % 2], add=True)
        for sc in scat:
            if sc is not None:
                sc.wait()

        plsc.subcore_barrier()
        pltpu.sync_copy(
            acc.at[pl.ds(sid * RPT, RPT)],
            out.at[cid, pl.ds(sid * RPT, RPT)],
        )

    return edge_kernel


def _k2_body(x_ref, w1_ref, degp_ref, g1_ref, dinv_ref):
    deg = degp_ref[0:N, :] + degp_ref[NPAD:NPAD + N, :] + 1.0
    dinv = lax.rsqrt(deg)
    h1 = jnp.dot(x_ref[...], w1_ref[...], preferred_element_type=jnp.float32)
    g1_ref[pl.ds(0, N), :] = h1 * dinv
    g1_ref[pl.ds(N, NPAD - N), :] = jnp.zeros((NPAD - N, H), jnp.float32)
    dinv_ref[...] = dinv


def _k4_body(p_ref, g1_ref, dinv_ref, b1_ref, w2_ref, g2_ref):
    dinv = dinv_ref[...]
    s = p_ref[0:N, :] + p_ref[NPAD:NPAD + N, :] + g1_ref[0:N, :]
    r1 = jnp.maximum(dinv * s + b1_ref[...], 0.0)
    h2 = jnp.dot(r1, w2_ref[...], preferred_element_type=jnp.float32)
    g2_ref[pl.ds(0, N), :] = h2 * dinv
    g2_ref[pl.ds(N, NPAD - N), :] = jnp.zeros((NPAD - N, CP), jnp.float32)


def _k6_body(q_ref, g2_ref, dinv_ref, b2_ref, w3_ref, b3_ref, w4_ref, b4_ref,
             out_ref):
    dinv = dinv_ref[...]
    s = q_ref[0:N, :] + q_ref[NPAD:NPAD + N, :] + g2_ref[0:N, :]
    agg2 = dinv * s + b2_ref[...]
    z1 = jnp.maximum(
        jnp.dot(agg2, w3_ref[...], preferred_element_type=jnp.float32)
        + b3_ref[...], 0.0)
    z = jnp.dot(z1, w4_ref[...], preferred_element_type=jnp.float32) \
        + b4_ref[...]
    m = jnp.max(z, axis=1, keepdims=True)
    lse = jnp.log(jnp.sum(jnp.exp(z - m), axis=1, keepdims=True)) + m
    out_ref[...] = z - lse


def kernel(x, edge_index, W1, b1, W2, b2, W3, b3, W4, b4):
    deg_k = _make_deg_kernel()
    edge16 = _make_edge_kernel(H)
    edge8 = _make_edge_kernel(CP)

    ei_flat = edge_index.reshape(2 * E)

    degp = deg_k(ei_flat).reshape(NC * NPAD, 1)

    g1, dinv = pl.pallas_call(
        _k2_body,
        out_shape=[
            jax.ShapeDtypeStruct((NPAD, H), jnp.float32),
            jax.ShapeDtypeStruct((N, 1), jnp.float32),
        ],
    )(x, W1, degp)

    z16 = jnp.zeros((NPAD, H), jnp.float32)
    p1 = edge16(ei_flat, g1, z16).reshape(NC * NPAD, H)

    W2p = jnp.concatenate([W2, jnp.zeros((H, CP - C), jnp.float32)], axis=1)
    g2 = pl.pallas_call(
        _k4_body,
        out_shape=jax.ShapeDtypeStruct((NPAD, CP), jnp.float32),
    )(p1, g1, dinv, b1.reshape(1, H), W2p)

    z8 = jnp.zeros((NPAD, CP), jnp.float32)
    p2 = edge8(ei_flat, g2, z8).reshape(NC * NPAD, CP)

    b2p = jnp.concatenate([b2, jnp.zeros((CP - C,), jnp.float32)])
    W3p = jnp.concatenate([W3, jnp.zeros((CP - C, 32), jnp.float32)], axis=0)
    out = pl.pallas_call(
        _k6_body,
        out_shape=jax.ShapeDtypeStruct((N, C), jnp.float32),
    )(p2, g2, dinv, b2p.reshape(1, CP), W3p, b3.reshape(1, 32), W4,
      b4.reshape(1, C))
    return out


# speedup vs baseline: 1.2146x; 1.0126x over previous
import functools

import jax
import jax.numpy as jnp
from jax import lax
from jax.experimental import pallas as pl
from jax.experimental.pallas import tpu as pltpu, tpu_sc as plsc

N = 10000
E = 320000
D = 128
H = 16
C = 5
CP = 8

NPAD = 10240
NC = 2
NS = 16
NW = NC * NS
EW = E // NW
CHUNKS = (2504, 2504, 2504, 2488)
OFFS = (0, 2504, 5008, 7512)
NITER = len(CHUNKS)
CMAX = CHUNKS[0]
RPT = NPAD // NS


def _fill(ref, n, val):
    v = jnp.full((16,), val, jnp.float32)

    def body(i, c):
        ref[pl.ds(i * 16, 16)] = v
        return c

    lax.fori_loop(0, n // 16, body, 0)


def _make_deg_kernel():
    mesh = plsc.VectorSubcoreMesh(core_axis_name="c", subcore_axis_name="s")

    @functools.partial(
        pl.kernel,
        mesh=mesh,
        out_type=jax.ShapeDtypeStruct((NC, NPAD), jnp.float32),
        scratch_types=[
            pltpu.VMEM((EW,), jnp.int32),
            pltpu.VMEM((EW,), jnp.float32),
            pltpu.VMEM((RPT,), jnp.float32),
            pltpu.VMEM_SHARED((NPAD,), jnp.float32),
            pltpu.SemaphoreType.DMA,
        ],
    )
    def deg_kernel(ei, out, dst_v, ones_v, z_v, acc, sem):
        cid = lax.axis_index("c")
        sid = lax.axis_index("s")
        wid = cid * NS + sid
        base = wid * EW

        idx_cp = pltpu.async_copy(ei.at[pl.ds(E + base, EW)], dst_v, sem)
        _fill(z_v, RPT, 0.0)
        _fill(ones_v, EW, 1.0)
        pltpu.sync_copy(z_v, acc.at[pl.ds(sid * RPT, RPT)])
        plsc.subcore_barrier()
        idx_cp.wait()
        pltpu.sync_copy(ones_v, acc.at[dst_v], add=True)
        plsc.subcore_barrier()
        pltpu.sync_copy(
            acc.at[pl.ds(sid * RPT, RPT)],
            out.at[cid, pl.ds(sid * RPT, RPT)],
        )

    return deg_kernel


def _make_edge_kernel(width):
    mesh = plsc.VectorSubcoreMesh(core_axis_name="c", subcore_axis_name="s")

    @functools.partial(
        pl.kernel,
        mesh=mesh,
        out_type=jax.ShapeDtypeStruct((NC, NPAD, width), jnp.float32),
        scratch_types=[
            [pltpu.VMEM((c,), jnp.int32) for c in CHUNKS],
            [pltpu.VMEM((c,), jnp.int32) for c in CHUNKS],
            [pltpu.VMEM((CMAX, width), jnp.float32) for _ in range(2)],
            pltpu.VMEM_SHARED((NPAD, width), jnp.float32),
            pltpu.SemaphoreType.DMA,
            [pltpu.SemaphoreType.DMA for _ in range(2)],
            [pltpu.SemaphoreType.DMA for _ in range(2)],
        ],
        compiler_params=pltpu.CompilerParams(use_tc_tiling_on_sc=False),
    )
    def edge_kernel(ei, g, zeros, out, srcs, dsts, rows, acc, semi, semg,
                    sems):
        cid = lax.axis_index("c")
        sid = lax.axis_index("s")
        wid = cid * NS + sid
        base = wid * EW

        src0_cp = pltpu.async_copy(
            ei.at[pl.ds(base, CHUNKS[0])], srcs[0], semi)
        cps = [pltpu.async_copy(
            zeros.at[pl.ds(sid * RPT, RPT)],
            acc.at[pl.ds(sid * RPT, RPT)], semi)]
        for j in range(NITER):
            off = base + OFFS[j]
            if j > 0:
                cps.append(pltpu.async_copy(ei.at[pl.ds(off, CHUNKS[j])],
                                            srcs[j], semi))
            cps.append(pltpu.async_copy(ei.at[pl.ds(E + off, CHUNKS[j])],
                                        dsts[j], semi))

        def buf(j):
            b = rows[j % 2]
            return b if CHUNKS[j] == CMAX else b.at[pl.ds(0, CHUNKS[j])]

        src0_cp.wait()
        gathers = [None, None]
        gathers[0] = pltpu.async_copy(g.at[srcs[0]], buf(0), semg[0])
        for cp in cps:
            cp.wait()
        plsc.subcore_barrier()

        scat = [None, None]
        for j in range(NITER):
            gathers[j % 2].wait()
            if j + 1 < NITER:
                nb = (j + 1) % 2
                if scat[nb] is not None:
                    scat[nb].wait()
                    scat[nb] = None
                gathers[nb] = pltpu.async_copy(
                    g.at[srcs[j + 1]], buf(j + 1), semg[nb])
            scat[j % 2] = pltpu.async_copy(
                buf(j), acc.at[dsts[j]], sems[j % 2], add=True)
        for sc in scat:
            if sc is not None:
                sc.wait()

        plsc.subcore_barrier()
        pltpu.sync_copy(
            acc.at[pl.ds(sid * RPT, RPT)],
            out.at[cid, pl.ds(sid * RPT, RPT)],
        )

    return edge_kernel


def _k2a_body(x_ref, w1_ref, h1_ref):
    h1_ref[...] = jnp.dot(x_ref[...], w1_ref[...],
                          preferred_element_type=jnp.float32)


def _k2_body(h1_ref, degp_ref, g1_ref, dinv_ref):
    deg = degp_ref[0:N, :] + degp_ref[NPAD:NPAD + N, :] + 1.0
    dinv = lax.rsqrt(deg)
    g1_ref[pl.ds(0, N), :] = h1_ref[...] * dinv
    g1_ref[pl.ds(N, NPAD - N), :] = jnp.zeros((NPAD - N, H), jnp.float32)
    dinv_ref[...] = dinv


def _k4_body(p_ref, g1_ref, dinv_ref, b1_ref, w2_ref, g2_ref):
    dinv = dinv_ref[...]
    s = p_ref[0:N, :] + p_ref[NPAD:NPAD + N, :] + g1_ref[0:N, :]
    r1 = jnp.maximum(dinv * s + b1_ref[...], 0.0)
    h2 = jnp.dot(r1, w2_ref[...], preferred_element_type=jnp.float32)
    g2_ref[pl.ds(0, N), :] = h2 * dinv
    g2_ref[pl.ds(N, NPAD - N), :] = jnp.zeros((NPAD - N, CP), jnp.float32)


def _k6_body(q_ref, g2_ref, dinv_ref, b2_ref, w3_ref, b3_ref, w4_ref, b4_ref,
             out_ref):
    dinv = dinv_ref[...]
    s = q_ref[0:N, :] + q_ref[NPAD:NPAD + N, :] + g2_ref[0:N, :]
    agg2 = dinv * s + b2_ref[...]
    z1 = jnp.maximum(
        jnp.dot(agg2, w3_ref[...], preferred_element_type=jnp.float32)
        + b3_ref[...], 0.0)
    z = jnp.dot(z1, w4_ref[...], preferred_element_type=jnp.float32) \
        + b4_ref[...]
    m = jnp.max(z, axis=1, keepdims=True)
    lse = jnp.log(jnp.sum(jnp.exp(z - m), axis=1, keepdims=True)) + m
    out_ref[...] = z - lse


def kernel(x, edge_index, W1, b1, W2, b2, W3, b3, W4, b4):
    deg_k = _make_deg_kernel()
    edge16 = _make_edge_kernel(H)
    edge8 = _make_edge_kernel(CP)

    ei_flat = edge_index.reshape(2 * E)

    h1 = pl.pallas_call(
        _k2a_body,
        out_shape=jax.ShapeDtypeStruct((N, H), jnp.float32),
    )(x, W1)

    degp = deg_k(ei_flat).reshape(NC * NPAD, 1)

    g1, dinv = pl.pallas_call(
        _k2_body,
        out_shape=[
            jax.ShapeDtypeStruct((NPAD, H), jnp.float32),
            jax.ShapeDtypeStruct((N, 1), jnp.float32),
        ],
    )(h1, degp)

    z16 = jnp.zeros((NPAD, H), jnp.float32)
    p1 = edge16(ei_flat, g1, z16).reshape(NC * NPAD, H)

    W2p = jnp.concatenate([W2, jnp.zeros((H, CP - C), jnp.float32)], axis=1)
    g2 = pl.pallas_call(
        _k4_body,
        out_shape=jax.ShapeDtypeStruct((NPAD, CP), jnp.float32),
    )(p1, g1, dinv, b1.reshape(1, H), W2p)

    z8 = jnp.zeros((NPAD, CP), jnp.float32)
    p2 = edge8(ei_flat, g2, z8).reshape(NC * NPAD, CP)

    b2p = jnp.concatenate([b2, jnp.zeros((CP - C,), jnp.float32)])
    W3p = jnp.concatenate([W3, jnp.zeros((CP - C, 32), jnp.float32)], axis=0)
    out = pl.pallas_call(
        _k6_body,
        out_shape=jax.ShapeDtypeStruct((N, C), jnp.float32),
    )(p2, g2, dinv, b2p.reshape(1, CP), W3p, b3.reshape(1, 32), W4,
      b4.reshape(1, C))
    return out


# speedup vs baseline: 1.3301x; 1.0951x over previous
import functools

import jax
import jax.numpy as jnp
from jax import lax
from jax.experimental import pallas as pl
from jax.experimental.pallas import tpu as pltpu, tpu_sc as plsc

N = 10000
E = 320000
D = 128
H = 16
C = 5
CP = 8

NPAD = 10240
NC = 2
NS = 16
NW = NC * NS
EW = E // NW
CHUNKS = (2504, 2504, 2504, 2488)
OFFS = (0, 2504, 5008, 7512)
NITER = len(CHUNKS)
CMAX = CHUNKS[0]
RPT = NPAD // NS


def _fill(ref, n, val):
    v = jnp.full((16,), val, jnp.float32)

    def body(i, c):
        ref[pl.ds(i * 16, 16)] = v
        return c

    lax.fori_loop(0, n // 16, body, 0)


def _make_deg_kernel():
    mesh = plsc.VectorSubcoreMesh(core_axis_name="c", subcore_axis_name="s")

    @functools.partial(
        pl.kernel,
        mesh=mesh,
        out_type=jax.ShapeDtypeStruct((NC, NPAD), jnp.float32),
        scratch_types=[
            pltpu.VMEM((EW,), jnp.int32),
            pltpu.VMEM((EW,), jnp.float32),
            pltpu.VMEM((RPT,), jnp.float32),
            pltpu.VMEM_SHARED((NPAD,), jnp.float32),
            pltpu.SemaphoreType.DMA,
        ],
    )
    def deg_kernel(ei, out, dst_v, ones_v, z_v, acc, sem):
        cid = lax.axis_index("c")
        sid = lax.axis_index("s")
        wid = cid * NS + sid
        base = wid * EW

        idx_cp = pltpu.async_copy(ei.at[pl.ds(E + base, EW)], dst_v, sem)
        _fill(z_v, RPT, 0.0)
        _fill(ones_v, EW, 1.0)
        pltpu.sync_copy(z_v, acc.at[pl.ds(sid * RPT, RPT)])
        plsc.subcore_barrier()
        idx_cp.wait()
        pltpu.sync_copy(ones_v, acc.at[dst_v], add=True)
        plsc.subcore_barrier()
        pltpu.sync_copy(
            acc.at[pl.ds(sid * RPT, RPT)],
            out.at[cid, pl.ds(sid * RPT, RPT)],
        )

    return deg_kernel


def _make_edge_kernel(width):
    mesh = plsc.VectorSubcoreMesh(core_axis_name="c", subcore_axis_name="s")

    @functools.partial(
        pl.kernel,
        mesh=mesh,
        out_type=jax.ShapeDtypeStruct((NC, NPAD, width), jnp.float32),
        scratch_types=[
            [pltpu.VMEM((c,), jnp.int32) for c in CHUNKS],
            [pltpu.VMEM((c,), jnp.int32) for c in CHUNKS],
            [pltpu.VMEM((CMAX, width), jnp.float32) for _ in range(2)],
            pltpu.VMEM_SHARED((NPAD, width), jnp.float32),
            pltpu.SemaphoreType.DMA,
            [pltpu.SemaphoreType.DMA for _ in range(2)],
            [pltpu.SemaphoreType.DMA for _ in range(2)],
        ],
        compiler_params=pltpu.CompilerParams(use_tc_tiling_on_sc=False),
    )
    def edge_kernel(ei, g, zeros, out, srcs, dsts, rows, acc, semi, semg,
                    sems):
        cid = lax.axis_index("c")
        sid = lax.axis_index("s")
        wid = cid * NS + sid
        base = wid * EW

        src0_cp = pltpu.async_copy(
            ei.at[pl.ds(base, CHUNKS[0])], srcs[0], semi)
        cps = [pltpu.async_copy(
            zeros.at[pl.ds(sid * RPT, RPT)],
            acc.at[pl.ds(sid * RPT, RPT)], semi)]
        for j in range(NITER):
            off = base + OFFS[j]
            if j > 0:
                cps.append(pltpu.async_copy(ei.at[pl.ds(off, CHUNKS[j])],
                                            srcs[j], semi))
            cps.append(pltpu.async_copy(ei.at[pl.ds(E + off, CHUNKS[j])],
                                        dsts[j], semi))

        def buf(j):
            b = rows[j % 2]
            return b if CHUNKS[j] == CMAX else b.at[pl.ds(0, CHUNKS[j])]

        src0_cp.wait()
        gathers = [None, None]
        gathers[0] = pltpu.async_copy(g.at[srcs[0]], buf(0), semg[0])
        for cp in cps:
            cp.wait()
        plsc.subcore_barrier()

        scat = [None, None]
        for j in range(NITER):
            gathers[j % 2].wait()
            if j + 1 < NITER:
                nb = (j + 1) % 2
                if scat[nb] is not None:
                    scat[nb].wait()
                    scat[nb] = None
                gathers[nb] = pltpu.async_copy(
                    g.at[srcs[j + 1]], buf(j + 1), semg[nb])
            scat[j % 2] = pltpu.async_copy(
                buf(j), acc.at[dsts[j]], sems[j % 2], add=True)
        for sc in scat:
            if sc is not None:
                sc.wait()

        plsc.subcore_barrier()
        pltpu.sync_copy(
            acc.at[pl.ds(sid * RPT, RPT)],
            out.at[cid, pl.ds(sid * RPT, RPT)],
        )

    return edge_kernel


def _k2a_body(x_ref, w1_ref, h1_ref):
    h1_ref[...] = jnp.dot(x_ref[...], w1_ref[...],
                          preferred_element_type=jnp.float32)


def _k6_body(q_ref, g2_ref, dinv_ref, w2_ref, b2_ref, w3_ref, b3_ref,
             w4_ref, b4_ref, out_ref):
    dinv = dinv_ref[...]
    s2 = dinv * (q_ref[0:N, :] + q_ref[NPAD:NPAD + N, :] + g2_ref[...])
    w23 = jnp.dot(w2_ref[...], w3_ref[...],
                  preferred_element_type=jnp.float32)
    b23 = jnp.dot(b2_ref[...], w3_ref[...],
                  preferred_element_type=jnp.float32) + b3_ref[...]
    z1 = jnp.maximum(
        jnp.dot(s2, w23, preferred_element_type=jnp.float32) + b23, 0.0)
    z = jnp.dot(z1, w4_ref[...], preferred_element_type=jnp.float32) \
        + b4_ref[...]
    m = jnp.max(z, axis=1, keepdims=True)
    lse = jnp.log(jnp.sum(jnp.exp(z - m), axis=1, keepdims=True)) + m
    out_ref[...] = z - lse


def kernel(x, edge_index, W1, b1, W2, b2, W3, b3, W4, b4):
    deg_k = _make_deg_kernel()
    edge16 = _make_edge_kernel(H)

    ei_flat = edge_index.reshape(2 * E)

    h1 = pl.pallas_call(
        _k2a_body,
        out_shape=jax.ShapeDtypeStruct((N, H), jnp.float32),
    )(x, W1)

    degp = deg_k(ei_flat)

    deg = degp[0, :N] + degp[1, :N] + 1.0
    dinv = lax.rsqrt(deg)[:, None]
    g1 = h1 * dinv

    z16 = jnp.zeros((NPAD, H), jnp.float32)
    p1 = edge16(ei_flat, g1, z16)

    r1 = jnp.maximum(dinv * (p1[0, :N] + p1[1, :N] + g1) + b1[None, :], 0.0)
    g2 = r1 * dinv

    p2 = edge16(ei_flat, g2, z16).reshape(NC * NPAD, H)

    out = pl.pallas_call(
        _k6_body,
        out_shape=jax.ShapeDtypeStruct((N, C), jnp.float32),
    )(p2, g2, dinv, W2, b2.reshape(1, C), W3, b3.reshape(1, 32), W4,
      b4.reshape(1, C))
    return out
